# baseline (speedup 1.0000x reference)
import jax
import jax.numpy as jnp
from jax import lax
from jax.experimental import pallas as pl
from jax.experimental.pallas import tpu as pltpu

NUM_EXPERTS = 8
EXPERT_CAPACITY = 640
D_MODEL = 768
NUM_TOKENS = 4096
BLOCK_T = 256
EC = NUM_EXPERTS * EXPERT_CAPACITY


def _router_block(x_ref, w_ref, b_ref, out_ref, out2_ref, carry_ref):
    i = pl.program_id(0)

    @pl.when(i == 0)
    def _init():
        carry_ref[...] = jnp.zeros_like(carry_ref)

    x = x_ref[...]
    w = w_ref[...]
    logits = jax.lax.dot_general(
        x, w, (((1,), (0,)), ((), ())),
        preferred_element_type=jnp.float32,
    ) + b_ref[...]

    m = jnp.max(logits, axis=1, keepdims=True)
    s = jnp.sum(jnp.exp(logits - m), axis=1, keepdims=True)
    gate = 1.0 / s

    e_iota = lax.broadcasted_iota(jnp.int32, (BLOCK_T, NUM_EXPERTS), 1)
    is_max = logits == m
    idx = jnp.min(jnp.where(is_max, e_iota, NUM_EXPERTS), axis=1, keepdims=True)
    onehot = (e_iota == idx).astype(jnp.float32)

    r = lax.broadcasted_iota(jnp.int32, (BLOCK_T, BLOCK_T), 0)
    c = lax.broadcasted_iota(jnp.int32, (BLOCK_T, BLOCK_T), 1)
    tri = (r >= c).astype(jnp.float32)
    csum = jax.lax.dot_general(
        tri, onehot, (((1,), (0,)), ((), ())),
        preferred_element_type=jnp.float32,
    )

    carry = carry_ref[...]
    pos = jnp.sum(onehot * (csum + carry), axis=1, keepdims=True)
    carry_ref[...] = carry + jnp.sum(onehot, axis=0, keepdims=True)

    gate = gate * (pos < float(EXPERT_CAPACITY)).astype(jnp.float32)
    target = idx * EXPERT_CAPACITY + pos.astype(jnp.int32)

    lane = lax.broadcasted_iota(jnp.int32, (BLOCK_T, EC), 1)
    row = jnp.where(lane == target, gate, 0.0)
    out_ref[...] = row
    out2_ref[...] = row


def kernel(inputs, W, b):
    b2 = b.reshape(1, NUM_EXPERTS)
    out = pl.pallas_call(
        _router_block,
        grid=(NUM_TOKENS // BLOCK_T,),
        in_specs=[
            pl.BlockSpec((BLOCK_T, D_MODEL), lambda i: (i, 0)),
            pl.BlockSpec((D_MODEL, NUM_EXPERTS), lambda i: (0, 0)),
            pl.BlockSpec((1, NUM_EXPERTS), lambda i: (0, 0)),
        ],
        out_specs=[
            pl.BlockSpec((BLOCK_T, EC), lambda i: (i, 0)),
            pl.BlockSpec((BLOCK_T, EC), lambda i: (i, 0)),
        ],
        out_shape=[
            jax.ShapeDtypeStruct((NUM_TOKENS, EC), jnp.float32),
            jax.ShapeDtypeStruct((NUM_TOKENS, EC), jnp.float32),
        ],
        scratch_shapes=[pltpu.VMEM((1, NUM_EXPERTS), jnp.float32)],
    )(inputs, W, b2)
    dispatch = out[0].reshape(NUM_TOKENS, NUM_EXPERTS, EXPERT_CAPACITY)
    combined = out[1].reshape(NUM_TOKENS, NUM_EXPERTS, EXPERT_CAPACITY)
    return (dispatch, combined)


# speedup vs baseline: 3.3861x; 3.3861x over previous
import jax
import jax.numpy as jnp
from jax import lax
from jax.experimental import pallas as pl
from jax.experimental.pallas import tpu as pltpu

NUM_EXPERTS = 8
EXPERT_CAPACITY = 640
D_MODEL = 768
NUM_TOKENS = 4096
BLOCK_T = 256
EC = NUM_EXPERTS * EXPERT_CAPACITY


def _router_block(x_ref, w_ref, b_ref, out_ref, out2_ref, carry_ref):
    i = pl.program_id(0)

    @pl.when(i == 0)
    def _init():
        carry_ref[...] = jnp.zeros_like(carry_ref)

    x = x_ref[...]
    w = w_ref[...]
    logits = jax.lax.dot_general(
        x, w, (((1,), (0,)), ((), ())),
        preferred_element_type=jnp.float32,
    ) + b_ref[...]

    m = jnp.max(logits, axis=1, keepdims=True)
    s = jnp.sum(jnp.exp(logits - m), axis=1, keepdims=True)
    gate = 1.0 / s

    e_iota = lax.broadcasted_iota(jnp.int32, (BLOCK_T, NUM_EXPERTS), 1)
    is_max = logits == m
    idx = jnp.min(jnp.where(is_max, e_iota, NUM_EXPERTS), axis=1, keepdims=True)
    onehot = (e_iota == idx).astype(jnp.float32)

    r = lax.broadcasted_iota(jnp.int32, (BLOCK_T, BLOCK_T), 0)
    c = lax.broadcasted_iota(jnp.int32, (BLOCK_T, BLOCK_T), 1)
    tri = (r >= c).astype(jnp.float32)
    csum = jax.lax.dot_general(
        tri, onehot, (((1,), (0,)), ((), ())),
        preferred_element_type=jnp.float32,
    )

    carry = carry_ref[...]
    pos = jnp.sum(onehot * (csum + carry), axis=1, keepdims=True)
    carry_ref[...] = carry + jnp.sum(onehot, axis=0, keepdims=True)

    posi = pos.astype(jnp.int32).reshape(BLOCK_T, 1, 1)
    idx3 = idx.reshape(BLOCK_T, 1, 1)
    gate3 = gate.reshape(BLOCK_T, 1, 1)
    e3 = lax.broadcasted_iota(jnp.int32, (BLOCK_T, NUM_EXPERTS, EXPERT_CAPACITY), 1)
    c3 = lax.broadcasted_iota(jnp.int32, (BLOCK_T, NUM_EXPERTS, EXPERT_CAPACITY), 2)
    row = jnp.where((e3 == idx3) & (c3 == posi), gate3, 0.0)
    out_ref[...] = row
    out2_ref[...] = row


def kernel(inputs, W, b):
    b2 = b.reshape(1, NUM_EXPERTS)
    out = pl.pallas_call(
        _router_block,
        grid=(NUM_TOKENS // BLOCK_T,),
        in_specs=[
            pl.BlockSpec((BLOCK_T, D_MODEL), lambda i: (i, 0)),
            pl.BlockSpec((D_MODEL, NUM_EXPERTS), lambda i: (0, 0)),
            pl.BlockSpec((1, NUM_EXPERTS), lambda i: (0, 0)),
        ],
        out_specs=[
            pl.BlockSpec((BLOCK_T, NUM_EXPERTS, EXPERT_CAPACITY), lambda i: (i, 0, 0)),
            pl.BlockSpec((BLOCK_T, NUM_EXPERTS, EXPERT_CAPACITY), lambda i: (i, 0, 0)),
        ],
        out_shape=[
            jax.ShapeDtypeStruct((NUM_TOKENS, NUM_EXPERTS, EXPERT_CAPACITY), jnp.float32),
            jax.ShapeDtypeStruct((NUM_TOKENS, NUM_EXPERTS, EXPERT_CAPACITY), jnp.float32),
        ],
        scratch_shapes=[pltpu.VMEM((1, NUM_EXPERTS), jnp.float32)],
    )(inputs, W, b2)
    return (out[0], out[1])
